# baseline (speedup 1.0000x reference)
import jax
import jax.numpy as jnp
import numpy as np
from jax.experimental import pallas as pl
from jax.experimental.pallas import tpu as pltpu
from jax.experimental.pallas import tpu_sc as plsc

EPS = 1e-9
SCORE_TH = 0.5
BIG = 100000.0

NUM_ANCHORS_LIST = (6400, 1600, 400, 100)
N, M, A, NC = 8, 200, 8500, 80
NTILES = 32
CH = 256
NG = CH // 16
SC_END = NTILES * CH
TAIL_BLK = 512

_BOUNDS = [0, 6400, 8000, 8400, 8500]


def _sc_stats_body(scores_hbm, pred_hbm, out_hbm, sbuf, pbuf, accscr):
    c = jax.lax.axis_index("c")
    s = jax.lax.axis_index("s")
    tid = c * 16 + s
    lo = tid * CH

    for l in range(12):
        accscr[l] = jnp.zeros((16,), jnp.float32)

    iota16 = jax.lax.iota(jnp.int32, 16)
    zeros16 = jnp.zeros((16,), jnp.int32)

    for n in range(N):
        pltpu.sync_copy(scores_hbm.at[n, pl.ds(lo, CH), :], sbuf)
        pltpu.sync_copy(pred_hbm.at[n, pl.ds(lo, CH), :], pbuf)

        def g_body(g, carry, _sbuf=sbuf, _pbuf=pbuf):
            rows = g * 16 + iota16
            aidx = lo + rows
            mxs = [plsc.load_gather(_sbuf, [rows, jnp.full((16,), c, jnp.int32)])
                   for c in range(8)]
            for col in range(8, NC):
                v = plsc.load_gather(_sbuf, [rows, jnp.full((16,), col, jnp.int32)])
                mxs[col % 8] = jnp.maximum(mxs[col % 8], v)
            while len(mxs) > 1:
                mxs = [jnp.maximum(mxs[i], mxs[i + 1])
                       for i in range(0, len(mxs), 2)]
            flag = mxs[0] > SCORE_TH
            x1 = plsc.load_gather(_pbuf, [rows, zeros16])
            y1 = plsc.load_gather(_pbuf, [rows, jnp.full((16,), 1, jnp.int32)])
            x2 = plsc.load_gather(_pbuf, [rows, jnp.full((16,), 2, jnp.int32)])
            y2 = plsc.load_gather(_pbuf, [rows, jnp.full((16,), 3, jnp.int32)])
            w16 = x2 - x1
            h16 = y2 - y1
            for l in range(3):
                m_l = flag & (aidx >= _BOUNDS[l]) & (aidx < _BOUNDS[l + 1])
                accscr[l] = accscr[l] + jnp.where(m_l, 1.0, 0.0)
                accscr[4 + l] = accscr[4 + l] + jnp.where(m_l, w16, 0.0)
                accscr[8 + l] = accscr[8 + l] + jnp.where(m_l, h16, 0.0)
            return carry

        jax.lax.fori_loop(0, NG, g_body, 0)

    pltpu.sync_copy(accscr, out_hbm.at[tid])


def _sc_stats(pred_scores, pred_bboxes):
    mesh = plsc.VectorSubcoreMesh(core_axis_name="c", subcore_axis_name="s")
    k = pl.kernel(
        _sc_stats_body,
        out_type=jax.ShapeDtypeStruct((NTILES, 12, 16), jnp.float32),
        mesh=mesh,
        scratch_types=[
            pltpu.VMEM((CH, NC), jnp.float32),
            pltpu.VMEM((CH, 4), jnp.float32),
            pltpu.VMEM((12, 16), jnp.float32),
        ],
        compiler_params=pltpu.CompilerParams(needs_layout_passes=False),
    )
    return k(pred_scores, pred_bboxes)


def _combine_kernel(sc_ref, tc_ref, w_ref, h_ref):
    p = jnp.sum(sc_ref[...], axis=0)
    t = tc_ref[...]
    cnt = jnp.maximum(
        jnp.sum(p[0:4], axis=1, keepdims=True) + t[0:4], 1.0)
    w_ref[...] = (jnp.sum(p[4:8], axis=1, keepdims=True) + t[4:8]) / cnt * 0.9
    h_ref[...] = (jnp.sum(p[8:12], axis=1, keepdims=True) + t[8:12]) / cnt * 0.9


def _combine(sc_partials, tc_partials):
    return pl.pallas_call(
        _combine_kernel,
        out_shape=[jax.ShapeDtypeStruct((4, 1), jnp.float32),
                   jax.ShapeDtypeStruct((4, 1), jnp.float32)],
    )(sc_partials, tc_partials)


def _cost_kernel(n_grid_n, n_grid_a,
                 centers_t_ref, gt_ref, padm_ref, predt_ref, tailsc_ref,
                 tailpred_ref, cost_ref, tcp_ref, acc_cnt, acc_w, acc_h):
    n = pl.program_id(0)
    a = pl.program_id(1)

    cx = centers_t_ref[0:1, :]
    cy = centers_t_ref[1:2, :]
    pt = predt_ref[0]
    px1 = pt[0:1, :]
    py1 = pt[1:2, :]
    px2 = pt[2:3, :]
    py2 = pt[3:4, :]
    l_ = cx - px1
    t_ = cy - py1
    r_ = px2 - cx
    b_ = py2 - cy
    num = jnp.minimum(l_, r_) * jnp.minimum(t_, b_)
    den = jnp.maximum(l_, r_) * jnp.maximum(t_, b_)
    ratio = jnp.clip(num / jnp.maximum(den, EPS), 1e-12, 1.0)
    one_m_centerness = 1.0 - jnp.sqrt(ratio)

    g = gt_ref[0]
    gx1 = g[:, 0:1]
    gy1 = g[:, 1:2]
    gx2 = g[:, 2:3]
    gy2 = g[:, 3:4]
    padb = padm_ref[0] > 0.5

    d1 = cx - gx1
    d2 = cy - gy1
    d3 = gx2 - cx
    d4 = gy2 - cy
    mind = jnp.minimum(jnp.minimum(d1, d2), jnp.minimum(d3, d4))
    valid = (mind > EPS) & padb

    ix1 = jnp.maximum(gx1, px1)
    iy1 = jnp.maximum(gy1, py1)
    ix2 = jnp.minimum(gx2, px2)
    iy2 = jnp.minimum(gy2, py2)
    inter = jnp.maximum(ix2 - ix1, 0.0) * jnp.maximum(iy2 - iy1, 0.0)
    agE = (gx2 - gx1) * (gy2 - gy1) + EPS
    ap = (px2 - px1) * (py2 - py1)
    union = (agE + ap) - inter
    one_m_iou = 1.0 - inter / union

    cost_ref[0, 0] = jnp.where(valid, one_m_centerness, BIG)
    cost_ref[1, 0] = jnp.where(valid, one_m_iou, BIG)

    @pl.when((n == 0) & (a == 0))
    def _init():
        acc_cnt[...] = jnp.zeros_like(acc_cnt)
        acc_w[...] = jnp.zeros_like(acc_w)
        acc_h[...] = jnp.zeros_like(acc_h)

    @pl.when(a == 0)
    def _tail():
        s = tailsc_ref[0]
        maxv = jnp.max(s, axis=1, keepdims=True)
        flag_row = maxv.reshape(1, TAIL_BLK) > SCORE_TH
        tp = tailpred_ref[0]
        w_row = tp[2:3, :] - tp[0:1, :]
        h_row = tp[3:4, :] - tp[1:2, :]
        gidx = SC_END + jax.lax.broadcasted_iota(jnp.int32, (1, TAIL_BLK), 1)
        lev = jax.lax.broadcasted_iota(jnp.int32, (4, 1), 0)
        starts_c = sum((lev == i) * s0 for i, s0 in enumerate(_BOUNDS[:4]))
        ends_c = sum((lev == i) * e0 for i, e0 in enumerate(_BOUNDS[1:]))
        levmask = (gidx >= starts_c) & (gidx < ends_c)
        flag4 = levmask & flag_row
        acc_cnt[...] += flag4.astype(jnp.float32)
        acc_w[...] += jnp.where(flag4, w_row, 0.0)
        acc_h[...] += jnp.where(flag4, h_row, 0.0)

    @pl.when((n == n_grid_n - 1) & (a == n_grid_a - 1))
    def _final():
        c4 = jnp.sum(acc_cnt[...], axis=1, keepdims=True)
        w4 = jnp.sum(acc_w[...], axis=1, keepdims=True)
        h4 = jnp.sum(acc_h[...], axis=1, keepdims=True)
        tcp_ref[...] = jnp.concatenate([c4, w4, h4], axis=0)


def _cost(centers, gt_bboxes, pad_gt_mask, pred_bboxes, pred_scores):
    BLK = 2176
    n_a = (A + BLK - 1) // BLK
    tail_i = SC_END // TAIL_BLK
    centers_t = centers.T
    predt = jnp.transpose(pred_bboxes, (0, 2, 1))

    def kfn(*refs):
        return _cost_kernel(N, n_a, *refs)

    return pl.pallas_call(
        kfn,
        grid=(N, n_a),
        in_specs=[
            pl.BlockSpec((2, BLK), lambda n, a: (0, a)),
            pl.BlockSpec((1, M, 4), lambda n, a: (n, 0, 0)),
            pl.BlockSpec((1, M, 1), lambda n, a: (n, 0, 0)),
            pl.BlockSpec((1, 4, BLK), lambda n, a: (n, 0, a)),
            pl.BlockSpec((1, TAIL_BLK, NC), lambda n, a: (n, tail_i, 0)),
            pl.BlockSpec((1, 4, TAIL_BLK),
                         lambda n, a: (n, 0, tail_i)),
        ],
        out_specs=[
            pl.BlockSpec((2, 1, M, BLK), lambda n, a: (0, n, 0, a)),
            pl.BlockSpec((12, 1), lambda n, a: (0, 0)),
        ],
        out_shape=[
            jax.ShapeDtypeStruct((2, N, M, A), jnp.float32),
            jax.ShapeDtypeStruct((12, 1), jnp.float32),
        ],
        scratch_shapes=[
            pltpu.VMEM((4, TAIL_BLK), jnp.float32),
            pltpu.VMEM((4, TAIL_BLK), jnp.float32),
            pltpu.VMEM((4, TAIL_BLK), jnp.float32),
        ],
        compiler_params=pltpu.CompilerParams(
            dimension_semantics=("arbitrary", "arbitrary")),
    )(centers_t, gt_bboxes, pad_gt_mask, predt, pred_scores, predt)


def kernel(centers, num_anchors_list, gt_labels, gt_bboxes, pad_gt_mask,
           bg_index, pred_bboxes, pred_scores):
    cost, tc_partials = _cost(centers, gt_bboxes, pad_gt_mask, pred_bboxes,
                              pred_scores)
    sc_partials = _sc_stats(pred_scores, pred_bboxes)
    w_avg, h_avg = _combine(sc_partials, tc_partials)
    return cost, w_avg.reshape(4), h_avg.reshape(4)


# speedup vs baseline: 1.4496x; 1.4496x over previous
import jax
import jax.numpy as jnp
import numpy as np
from jax.experimental import pallas as pl
from jax.experimental.pallas import tpu as pltpu

EPS = 1e-9
SCORE_TH = 0.5
BIG = 100000.0

NUM_ANCHORS_LIST = (6400, 1600, 400, 100)


def _fused_kernel(starts, ends, n_grid_n, n_grid_a,
                  centers_t_ref, gt_ref, padm_ref, predt_ref,
                  scores_ref, cost_ref, w_ref, h_ref,
                  acc_cnt, acc_w, acc_h):
    n = pl.program_id(0)
    a = pl.program_id(1)
    blk = centers_t_ref.shape[1]
    nlev = len(starts)

    cx = centers_t_ref[0:1, :]
    cy = centers_t_ref[1:2, :]
    pt = predt_ref[0]
    px1 = pt[0:1, :]
    py1 = pt[1:2, :]
    px2 = pt[2:3, :]
    py2 = pt[3:4, :]
    w_row = px2 - px1
    h_row = py2 - py1
    l_ = cx - px1
    t_ = cy - py1
    r_ = px2 - cx
    b_ = py2 - cy
    num = jnp.minimum(l_, r_) * jnp.minimum(t_, b_)
    den = jnp.maximum(l_, r_) * jnp.maximum(t_, b_)
    ratio = jnp.clip(num / jnp.maximum(den, EPS), 1e-12, 1.0)
    one_m_centerness = 1.0 - jnp.sqrt(ratio)

    g = gt_ref[0]
    gx1 = g[:, 0:1]
    gy1 = g[:, 1:2]
    gx2 = g[:, 2:3]
    gy2 = g[:, 3:4]
    padb = padm_ref[0] > 0.5

    d1 = cx - gx1
    d2 = cy - gy1
    d3 = gx2 - cx
    d4 = gy2 - cy
    mind = jnp.minimum(jnp.minimum(d1, d2), jnp.minimum(d3, d4))
    valid = (mind > EPS) & padb

    ix1 = jnp.maximum(gx1, px1)
    iy1 = jnp.maximum(gy1, py1)
    ix2 = jnp.minimum(gx2, px2)
    iy2 = jnp.minimum(gy2, py2)
    inter = jnp.maximum(ix2 - ix1, 0.0) * jnp.maximum(iy2 - iy1, 0.0)
    agE = (gx2 - gx1) * (gy2 - gy1) + EPS
    ap = w_row * h_row
    union = (agE + ap) - inter
    one_m_iou = 1.0 - inter / union

    cost_ref[0, 0] = jnp.where(valid, one_m_centerness, BIG)
    cost_ref[1, 0] = jnp.where(valid, one_m_iou, BIG)

    @pl.when((n == 0) & (a == 0))
    def _init():
        acc_cnt[...] = jnp.zeros_like(acc_cnt)
        acc_w[...] = jnp.zeros_like(acc_w)
        acc_h[...] = jnp.zeros_like(acc_h)

    s = scores_ref[0]
    maxv = jnp.max(s, axis=1, keepdims=True)
    flag_row = maxv.reshape(1, blk) > SCORE_TH

    gidx = a * blk + jax.lax.broadcasted_iota(jnp.int32, (1, blk), 1)
    lev = jax.lax.broadcasted_iota(jnp.int32, (nlev, 1), 0)
    starts_c = sum((lev == i) * s0 for i, s0 in enumerate(starts))
    ends_c = sum((lev == i) * e0 for i, e0 in enumerate(ends))
    levmask = (gidx >= starts_c) & (gidx < ends_c)
    flag4 = levmask & flag_row
    acc_cnt[...] += flag4.astype(jnp.float32)
    acc_w[...] += jnp.where(flag4, w_row, 0.0)
    acc_h[...] += jnp.where(flag4, h_row, 0.0)

    @pl.when((n == n_grid_n - 1) & (a == n_grid_a - 1))
    def _final():
        cnt = jnp.maximum(
            jnp.sum(acc_cnt[...], axis=1, keepdims=True), 1.0)
        w_ref[...] = jnp.sum(acc_w[...], axis=1, keepdims=True) / cnt * 0.9
        h_ref[...] = jnp.sum(acc_h[...], axis=1, keepdims=True) / cnt * 0.9


def _run(centers, gt_bboxes, pad_gt_mask, pred_bboxes, pred_scores):
    A = centers.shape[0]
    N, M, _ = gt_bboxes.shape
    NC = pred_scores.shape[2]
    BLK = 2944
    n_a = (A + BLK - 1) // BLK

    bounds = np.concatenate([[0], np.cumsum(NUM_ANCHORS_LIST)])
    starts = tuple(int(x) for x in bounds[:-1])
    ends = tuple(int(x) for x in bounds[1:])

    centers_t = centers.T
    predt = jnp.transpose(pred_bboxes, (0, 2, 1))

    def kfn(*refs):
        return _fused_kernel(starts, ends, N, n_a, *refs)

    cost, w_avg, h_avg = pl.pallas_call(
        kfn,
        grid=(N, n_a),
        in_specs=[
            pl.BlockSpec((2, BLK), lambda n, a: (0, a)),
            pl.BlockSpec((1, M, 4), lambda n, a: (n, 0, 0)),
            pl.BlockSpec((1, M, 1), lambda n, a: (n, 0, 0)),
            pl.BlockSpec((1, 4, BLK), lambda n, a: (n, 0, a)),
            pl.BlockSpec((1, BLK, NC), lambda n, a: (n, a, 0)),
        ],
        out_specs=[
            pl.BlockSpec((2, 1, M, BLK), lambda n, a: (0, n, 0, a)),
            pl.BlockSpec((4, 1), lambda n, a: (0, 0)),
            pl.BlockSpec((4, 1), lambda n, a: (0, 0)),
        ],
        out_shape=[
            jax.ShapeDtypeStruct((2, N, M, A), jnp.float32),
            jax.ShapeDtypeStruct((4, 1), jnp.float32),
            jax.ShapeDtypeStruct((4, 1), jnp.float32),
        ],
        scratch_shapes=[
            pltpu.VMEM((4, BLK), jnp.float32),
            pltpu.VMEM((4, BLK), jnp.float32),
            pltpu.VMEM((4, BLK), jnp.float32),
        ],
        compiler_params=pltpu.CompilerParams(
            dimension_semantics=("arbitrary", "arbitrary")),
    )(centers_t, gt_bboxes, pad_gt_mask, predt, pred_scores)
    return cost, w_avg.reshape(4), h_avg.reshape(4)


def kernel(centers, num_anchors_list, gt_labels, gt_bboxes, pad_gt_mask,
           bg_index, pred_bboxes, pred_scores):
    return _run(centers, gt_bboxes, pad_gt_mask, pred_bboxes, pred_scores)


# speedup vs baseline: 1.5050x; 1.0383x over previous
import jax
import jax.numpy as jnp
import numpy as np
from jax.experimental import pallas as pl
from jax.experimental.pallas import tpu as pltpu

EPS = 1e-9
SCORE_TH = 0.5
BIG = 100000.0

NUM_ANCHORS_LIST = (6400, 1600, 400, 100)


def _fused_kernel(starts, ends, n_grid_n, n_grid_a,
                  centers_t_ref, gt_ref, padm_ref, predt_ref,
                  scores_ref, cost_ref, w_ref, h_ref,
                  acc_cnt, acc_w, acc_h):
    n = pl.program_id(0)
    a = pl.program_id(1)
    blk = centers_t_ref.shape[1]
    nlev = len(starts)

    cx = centers_t_ref[0:1, :]
    cy = centers_t_ref[1:2, :]
    pt = predt_ref[0]
    px1 = pt[0:1, :]
    py1 = pt[1:2, :]
    px2 = pt[2:3, :]
    py2 = pt[3:4, :]
    w_row = px2 - px1
    h_row = py2 - py1
    l_ = cx - px1
    t_ = cy - py1
    r_ = px2 - cx
    b_ = py2 - cy
    num = jnp.minimum(l_, r_) * jnp.minimum(t_, b_)
    den = jnp.maximum(l_, r_) * jnp.maximum(t_, b_)
    ratio = jnp.clip(num / jnp.maximum(den, EPS), 1e-12, 1.0)
    one_m_centerness = 1.0 - jnp.sqrt(ratio)

    g = gt_ref[0]
    gx1 = g[:, 0:1]
    gy1 = g[:, 1:2]
    gx2 = g[:, 2:3]
    gy2 = g[:, 3:4]
    padb = padm_ref[0] > 0.5

    d1 = cx - gx1
    d2 = cy - gy1
    d3 = gx2 - cx
    d4 = gy2 - cy
    mind = jnp.minimum(jnp.minimum(d1, d2), jnp.minimum(d3, d4))
    valid = (mind > EPS) & padb

    ix1 = jnp.maximum(gx1, px1)
    iy1 = jnp.maximum(gy1, py1)
    ix2 = jnp.minimum(gx2, px2)
    iy2 = jnp.minimum(gy2, py2)
    inter = jnp.maximum(ix2 - ix1, 0.0) * jnp.maximum(iy2 - iy1, 0.0)
    agE = (gx2 - gx1) * (gy2 - gy1) + EPS
    ap = w_row * h_row
    union = (agE + ap) - inter
    one_m_iou = 1.0 - inter / union

    cost_ref[0, 0] = jnp.where(valid, one_m_centerness, BIG)
    cost_ref[1, 0] = jnp.where(valid, one_m_iou, BIG)

    @pl.when((n == 0) & (a == 0))
    def _init():
        acc_cnt[...] = jnp.zeros_like(acc_cnt)
        acc_w[...] = jnp.zeros_like(acc_w)
        acc_h[...] = jnp.zeros_like(acc_h)

    s = scores_ref[0]
    maxv = jnp.max(s, axis=1, keepdims=True)
    flag_row = maxv.reshape(1, blk) > SCORE_TH

    gidx = a * blk + jax.lax.broadcasted_iota(jnp.int32, (1, blk), 1)
    lev = jax.lax.broadcasted_iota(jnp.int32, (nlev, 1), 0)
    starts_c = sum((lev == i) * s0 for i, s0 in enumerate(starts))
    ends_c = sum((lev == i) * e0 for i, e0 in enumerate(ends))
    levmask = (gidx >= starts_c) & (gidx < ends_c)
    flag4 = levmask & flag_row
    acc_cnt[...] += flag4.astype(jnp.float32)
    acc_w[...] += jnp.where(flag4, w_row, 0.0)
    acc_h[...] += jnp.where(flag4, h_row, 0.0)

    @pl.when((n == n_grid_n - 1) & (a == n_grid_a - 1))
    def _final():
        cnt = jnp.maximum(
            jnp.sum(acc_cnt[...], axis=1, keepdims=True), 1.0)
        w_ref[...] = jnp.sum(acc_w[...], axis=1, keepdims=True) / cnt * 0.9
        h_ref[...] = jnp.sum(acc_h[...], axis=1, keepdims=True) / cnt * 0.9


def _run(centers, gt_bboxes, pad_gt_mask, pred_bboxes, pred_scores):
    A = centers.shape[0]
    N, M, _ = gt_bboxes.shape
    NC = pred_scores.shape[2]
    BLK = 2176
    n_a = (A + BLK - 1) // BLK

    bounds = np.concatenate([[0], np.cumsum(NUM_ANCHORS_LIST)])
    starts = tuple(int(x) for x in bounds[:-1])
    ends = tuple(int(x) for x in bounds[1:])

    centers_t = centers.T
    predt = jnp.transpose(pred_bboxes, (0, 2, 1))

    def kfn(*refs):
        return _fused_kernel(starts, ends, N, n_a, *refs)

    cost, w_avg, h_avg = pl.pallas_call(
        kfn,
        grid=(N, n_a),
        in_specs=[
            pl.BlockSpec((2, BLK), lambda n, a: (0, a)),
            pl.BlockSpec((1, M, 4), lambda n, a: (n, 0, 0)),
            pl.BlockSpec((1, M, 1), lambda n, a: (n, 0, 0)),
            pl.BlockSpec((1, 4, BLK), lambda n, a: (n, 0, a)),
            pl.BlockSpec((1, BLK, NC), lambda n, a: (n, a, 0)),
        ],
        out_specs=[
            pl.BlockSpec((2, 1, M, BLK), lambda n, a: (0, n, 0, a)),
            pl.BlockSpec((4, 1), lambda n, a: (0, 0)),
            pl.BlockSpec((4, 1), lambda n, a: (0, 0)),
        ],
        out_shape=[
            jax.ShapeDtypeStruct((2, N, M, A), jnp.float32),
            jax.ShapeDtypeStruct((4, 1), jnp.float32),
            jax.ShapeDtypeStruct((4, 1), jnp.float32),
        ],
        scratch_shapes=[
            pltpu.VMEM((4, BLK), jnp.float32),
            pltpu.VMEM((4, BLK), jnp.float32),
            pltpu.VMEM((4, BLK), jnp.float32),
        ],
        compiler_params=pltpu.CompilerParams(
            dimension_semantics=("arbitrary", "arbitrary")),
    )(centers_t, gt_bboxes, pad_gt_mask, predt, pred_scores)
    return cost, w_avg.reshape(4), h_avg.reshape(4)


def kernel(centers, num_anchors_list, gt_labels, gt_bboxes, pad_gt_mask,
           bg_index, pred_bboxes, pred_scores):
    return _run(centers, gt_bboxes, pad_gt_mask, pred_bboxes, pred_scores)
